# Initial kernel scaffold; baseline (speedup 1.0000x reference)
import jax
import jax.numpy as jnp
from jax.experimental import pallas as pl


def kernel(xyz1, xyz2, points1, points2, W0, b0, g0, beta0, W1, b1, g1, beta1):
    raise NotImplementedError("write your pallas kernel here")


# speedup vs baseline: 25.9524x; 25.9524x over previous
import functools

import jax
import jax.numpy as jnp
from jax.experimental import pallas as pl


def _stageA_body(x1t_ref, x2t_ref, p1_ref, p2_ref, w0_ref,
                 y0_ref, s0_ref, q0_ref):
    T = x1t_ref.shape[2]
    N2 = x2t_ref.shape[2]
    q = x1t_ref[0]
    k = x2t_ref[0]
    qk = jax.lax.dot_general(q, k, (((0,), (0,)), ((), ())),
                             preferred_element_type=jnp.float32)
    q2 = jnp.sum(q * q, axis=0)[:, None]
    k2 = jnp.sum(k * k, axis=0)[None, :]
    d2 = jnp.maximum(q2 + k2 - 2.0 * qk, 1e-12)

    col = jax.lax.broadcasted_iota(jnp.int32, (T, N2), 1)
    idxs = []
    ws = []
    for _ in range(3):
        m = jnp.min(d2, axis=1, keepdims=True)
        idx = jnp.min(jnp.where(d2 == m, col, N2), axis=1)
        w = jax.lax.rsqrt(m[:, 0])
        idxs.append(idx)
        ws.append(w)
        d2 = jnp.where(col == idx[:, None], jnp.inf, d2)
    wsum = ws[0] + ws[1] + ws[2]
    row = jax.lax.broadcasted_iota(jnp.int32, (N2, T), 0)
    wsT = jnp.zeros((N2, T), jnp.float32)
    for i in range(3):
        wn = ws[i] / wsum
        wsT = jnp.where(row == idxs[i][None, :], wn[None, :], wsT)

    interp = jnp.dot(p2_ref[0], wsT, preferred_element_type=jnp.float32)
    x_top = p1_ref[0]
    C1 = x_top.shape[0]
    y0 = (jnp.dot(w0_ref[:, :C1], x_top, preferred_element_type=jnp.float32)
          + jnp.dot(w0_ref[:, C1:], interp, preferred_element_type=jnp.float32))
    y0_ref[0] = y0
    s = jnp.sum(y0, axis=1)[None, :]
    sq = jnp.sum(y0 * y0, axis=1)[None, :]

    first = (pl.program_id(0) == 0) & (pl.program_id(1) == 0)

    @pl.when(first)
    def _():
        s0_ref[...] = s
        q0_ref[...] = sq

    @pl.when(jnp.logical_not(first))
    def _():
        s0_ref[...] += s
        q0_ref[...] += sq


def _stageB_body(y0_ref, s0_ref, q0_ref, g0_ref, bt0_ref, w1_ref,
                 y1_ref, s1_ref, q1_ref, *, count):
    eps = 1e-5
    mean = (s0_ref[0] / count)[:, None]
    var = (q0_ref[0] / count)[:, None] - mean * mean
    scale = g0_ref[0][:, None] * jax.lax.rsqrt(var + eps)
    shift = bt0_ref[0][:, None] - mean * scale
    h = jnp.maximum(y0_ref[0] * scale + shift, 0.0)
    y1 = jnp.dot(w1_ref[...], h, preferred_element_type=jnp.float32)
    y1_ref[0] = y1
    s = jnp.sum(y1, axis=1)[None, :]
    sq = jnp.sum(y1 * y1, axis=1)[None, :]

    first = (pl.program_id(0) == 0) & (pl.program_id(1) == 0)

    @pl.when(first)
    def _():
        s1_ref[...] = s
        q1_ref[...] = sq

    @pl.when(jnp.logical_not(first))
    def _():
        s1_ref[...] += s
        q1_ref[...] += sq


def _stageC_body(y1_ref, s1_ref, q1_ref, g1_ref, bt1_ref, out_ref, *, count):
    eps = 1e-5
    mean = (s1_ref[0] / count)[:, None]
    var = (q1_ref[0] / count)[:, None] - mean * mean
    scale = g1_ref[0][:, None] * jax.lax.rsqrt(var + eps)
    shift = bt1_ref[0][:, None] - mean * scale
    out_ref[0] = jnp.maximum(y1_ref[0] * scale + shift, 0.0)


def kernel(xyz1, xyz2, points1, points2, W0, b0, g0, beta0, W1, b1, g1, beta1):
    B, N1, _ = xyz1.shape
    N2 = xyz2.shape[1]
    C1 = points1.shape[1]
    C2 = points2.shape[1]
    CO = W0.shape[0]
    del b0, b1

    x1t = jnp.transpose(xyz1, (0, 2, 1))
    x2t = jnp.transpose(xyz2, (0, 2, 1))

    T = min(256, N1)
    NT = N1 // T
    count = float(B * N1)

    y0, s0, q0 = pl.pallas_call(
        _stageA_body,
        grid=(B, NT),
        in_specs=[
            pl.BlockSpec((1, 3, T), lambda b, i: (b, 0, i)),
            pl.BlockSpec((1, 3, N2), lambda b, i: (b, 0, 0)),
            pl.BlockSpec((1, C1, T), lambda b, i: (b, 0, i)),
            pl.BlockSpec((1, C2, N2), lambda b, i: (b, 0, 0)),
            pl.BlockSpec((CO, C1 + C2), lambda b, i: (0, 0)),
        ],
        out_specs=[
            pl.BlockSpec((1, CO, T), lambda b, i: (b, 0, i)),
            pl.BlockSpec((1, CO), lambda b, i: (0, 0)),
            pl.BlockSpec((1, CO), lambda b, i: (0, 0)),
        ],
        out_shape=[
            jax.ShapeDtypeStruct((B, CO, N1), jnp.float32),
            jax.ShapeDtypeStruct((1, CO), jnp.float32),
            jax.ShapeDtypeStruct((1, CO), jnp.float32),
        ],
    )(x1t, x2t, points1, points2, W0)

    T2 = min(512, N1)
    NT2 = N1 // T2
    y1, s1, q1 = pl.pallas_call(
        functools.partial(_stageB_body, count=count),
        grid=(B, NT2),
        in_specs=[
            pl.BlockSpec((1, CO, T2), lambda b, i: (b, 0, i)),
            pl.BlockSpec((1, CO), lambda b, i: (0, 0)),
            pl.BlockSpec((1, CO), lambda b, i: (0, 0)),
            pl.BlockSpec((1, CO), lambda b, i: (0, 0)),
            pl.BlockSpec((1, CO), lambda b, i: (0, 0)),
            pl.BlockSpec((CO, CO), lambda b, i: (0, 0)),
        ],
        out_specs=[
            pl.BlockSpec((1, CO, T2), lambda b, i: (b, 0, i)),
            pl.BlockSpec((1, CO), lambda b, i: (0, 0)),
            pl.BlockSpec((1, CO), lambda b, i: (0, 0)),
        ],
        out_shape=[
            jax.ShapeDtypeStruct((B, CO, N1), jnp.float32),
            jax.ShapeDtypeStruct((1, CO), jnp.float32),
            jax.ShapeDtypeStruct((1, CO), jnp.float32),
        ],
    )(y0, s0, q0, g0.reshape(1, CO), beta0.reshape(1, CO), W1)

    out = pl.pallas_call(
        functools.partial(_stageC_body, count=count),
        grid=(B, NT2),
        in_specs=[
            pl.BlockSpec((1, CO, T2), lambda b, i: (b, 0, i)),
            pl.BlockSpec((1, CO), lambda b, i: (0, 0)),
            pl.BlockSpec((1, CO), lambda b, i: (0, 0)),
            pl.BlockSpec((1, CO), lambda b, i: (0, 0)),
            pl.BlockSpec((1, CO), lambda b, i: (0, 0)),
        ],
        out_specs=pl.BlockSpec((1, CO, T2), lambda b, i: (b, 0, i)),
        out_shape=jax.ShapeDtypeStruct((B, CO, N1), jnp.float32),
    )(y1, s1, q1, g1.reshape(1, CO), beta1.reshape(1, CO))

    return out


# speedup vs baseline: 50.9912x; 1.9648x over previous
import functools

import jax
import jax.numpy as jnp
from jax.experimental import pallas as pl


def _stageA_body(x1t_ref, x2t_ref, p1_ref, p2_ref, w0_ref,
                 y0_ref, s0_ref, q0_ref):
    T = x1t_ref.shape[2]
    N2 = x2t_ref.shape[2]
    q = x1t_ref[0]
    k = x2t_ref[0]
    ktq = jax.lax.dot_general(k, q, (((0,), (0,)), ((), ())),
                              preferred_element_type=jnp.float32)
    q2 = jnp.sum(q * q, axis=0)[None, :]
    k2 = jnp.sum(k * k, axis=0)[:, None]
    sT = k2 - 2.0 * ktq

    ws = jnp.zeros((N2, T), jnp.float32)
    wsum = jnp.zeros((1, T), jnp.float32)
    for _ in range(3):
        m = jnp.min(sT, axis=0, keepdims=True)
        d2k = jnp.maximum(m + q2, 1e-12)
        w = jax.lax.rsqrt(d2k)
        eq = sT == m
        ws = jnp.where(eq, w, ws)
        sT = jnp.where(eq, jnp.inf, sT)
        wsum = wsum + w

    interp = jnp.dot(p2_ref[0], ws, preferred_element_type=jnp.float32)
    interp = interp / wsum
    x_top = p1_ref[0]
    C1 = x_top.shape[0]
    y0 = (jnp.dot(w0_ref[:, :C1], x_top, preferred_element_type=jnp.float32)
          + jnp.dot(w0_ref[:, C1:], interp, preferred_element_type=jnp.float32))
    y0_ref[0] = y0
    s = jnp.sum(y0, axis=1)[None, :]
    sq = jnp.sum(y0 * y0, axis=1)[None, :]

    first = (pl.program_id(0) == 0) & (pl.program_id(1) == 0)

    @pl.when(first)
    def _():
        s0_ref[...] = s
        q0_ref[...] = sq

    @pl.when(jnp.logical_not(first))
    def _():
        s0_ref[...] += s
        q0_ref[...] += sq


def _stageB_body(y0_ref, s0_ref, q0_ref, g0_ref, bt0_ref, w1_ref,
                 y1_ref, s1_ref, q1_ref, *, count):
    eps = 1e-5
    mean = (s0_ref[0] / count)[:, None]
    var = (q0_ref[0] / count)[:, None] - mean * mean
    scale = g0_ref[0][:, None] * jax.lax.rsqrt(var + eps)
    shift = bt0_ref[0][:, None] - mean * scale
    h = jnp.maximum(y0_ref[0] * scale + shift, 0.0)
    y1 = jnp.dot(w1_ref[...], h, preferred_element_type=jnp.float32)
    y1_ref[0] = y1
    s = jnp.sum(y1, axis=1)[None, :]
    sq = jnp.sum(y1 * y1, axis=1)[None, :]

    first = (pl.program_id(0) == 0) & (pl.program_id(1) == 0)

    @pl.when(first)
    def _():
        s1_ref[...] = s
        q1_ref[...] = sq

    @pl.when(jnp.logical_not(first))
    def _():
        s1_ref[...] += s
        q1_ref[...] += sq


def _stageC_body(y1_ref, s1_ref, q1_ref, g1_ref, bt1_ref, out_ref, *, count):
    eps = 1e-5
    mean = (s1_ref[0] / count)[:, None]
    var = (q1_ref[0] / count)[:, None] - mean * mean
    scale = g1_ref[0][:, None] * jax.lax.rsqrt(var + eps)
    shift = bt1_ref[0][:, None] - mean * scale
    out_ref[0] = jnp.maximum(y1_ref[0] * scale + shift, 0.0)


def kernel(xyz1, xyz2, points1, points2, W0, b0, g0, beta0, W1, b1, g1, beta1):
    B, N1, _ = xyz1.shape
    N2 = xyz2.shape[1]
    C1 = points1.shape[1]
    C2 = points2.shape[1]
    CO = W0.shape[0]
    del b0, b1

    x1t = jnp.transpose(xyz1, (0, 2, 1))
    x2t = jnp.transpose(xyz2, (0, 2, 1))

    T = min(512, N1)
    NT = N1 // T
    count = float(B * N1)

    y0, s0, q0 = pl.pallas_call(
        _stageA_body,
        grid=(B, NT),
        in_specs=[
            pl.BlockSpec((1, 3, T), lambda b, i: (b, 0, i)),
            pl.BlockSpec((1, 3, N2), lambda b, i: (b, 0, 0)),
            pl.BlockSpec((1, C1, T), lambda b, i: (b, 0, i)),
            pl.BlockSpec((1, C2, N2), lambda b, i: (b, 0, 0)),
            pl.BlockSpec((CO, C1 + C2), lambda b, i: (0, 0)),
        ],
        out_specs=[
            pl.BlockSpec((1, CO, T), lambda b, i: (b, 0, i)),
            pl.BlockSpec((1, CO), lambda b, i: (0, 0)),
            pl.BlockSpec((1, CO), lambda b, i: (0, 0)),
        ],
        out_shape=[
            jax.ShapeDtypeStruct((B, CO, N1), jnp.float32),
            jax.ShapeDtypeStruct((1, CO), jnp.float32),
            jax.ShapeDtypeStruct((1, CO), jnp.float32),
        ],
    )(x1t, x2t, points1, points2, W0)

    T2 = min(2048, N1)
    NT2 = N1 // T2
    y1, s1, q1 = pl.pallas_call(
        functools.partial(_stageB_body, count=count),
        grid=(B, NT2),
        in_specs=[
            pl.BlockSpec((1, CO, T2), lambda b, i: (b, 0, i)),
            pl.BlockSpec((1, CO), lambda b, i: (0, 0)),
            pl.BlockSpec((1, CO), lambda b, i: (0, 0)),
            pl.BlockSpec((1, CO), lambda b, i: (0, 0)),
            pl.BlockSpec((1, CO), lambda b, i: (0, 0)),
            pl.BlockSpec((CO, CO), lambda b, i: (0, 0)),
        ],
        out_specs=[
            pl.BlockSpec((1, CO, T2), lambda b, i: (b, 0, i)),
            pl.BlockSpec((1, CO), lambda b, i: (0, 0)),
            pl.BlockSpec((1, CO), lambda b, i: (0, 0)),
        ],
        out_shape=[
            jax.ShapeDtypeStruct((B, CO, N1), jnp.float32),
            jax.ShapeDtypeStruct((1, CO), jnp.float32),
            jax.ShapeDtypeStruct((1, CO), jnp.float32),
        ],
    )(y0, s0, q0, g0.reshape(1, CO), beta0.reshape(1, CO), W1)

    out = pl.pallas_call(
        functools.partial(_stageC_body, count=count),
        grid=(B, NT2),
        in_specs=[
            pl.BlockSpec((1, CO, T2), lambda b, i: (b, 0, i)),
            pl.BlockSpec((1, CO), lambda b, i: (0, 0)),
            pl.BlockSpec((1, CO), lambda b, i: (0, 0)),
            pl.BlockSpec((1, CO), lambda b, i: (0, 0)),
            pl.BlockSpec((1, CO), lambda b, i: (0, 0)),
        ],
        out_specs=pl.BlockSpec((1, CO, T2), lambda b, i: (b, 0, i)),
        out_shape=jax.ShapeDtypeStruct((B, CO, N1), jnp.float32),
    )(y1, s1, q1, g1.reshape(1, CO), beta1.reshape(1, CO))

    return out
